# baseline (speedup 1.0000x reference)
import functools

import jax
import jax.numpy as jnp
from jax import lax
from jax.experimental import pallas as pl
from jax.experimental.pallas import tpu as pltpu
from jax.experimental.pallas import tpu_sc as plsc

NUM_CLASSES = 1000
HIDDEN = 1024
BATCH = 4096

NC = 2
NS = 16
NW = NC * NS
B_PER_W = BATCH // NW
CHUNK = 16
NCHUNKS = B_PER_W // CHUNK
NBUF = 4
SBUF = 3


def _make_kernel():
  mesh = plsc.VectorSubcoreMesh(
      core_axis_name="c", subcore_axis_name="s", num_cores=NC,
      num_subcores=NS)

  @functools.partial(
      pl.kernel,
      out_type=jax.ShapeDtypeStruct((BATCH, HIDDEN), jnp.float32),
      mesh=mesh,
      scratch_types=[
          pltpu.VMEM((B_PER_W,), jnp.int32),
          [pltpu.VMEM((CHUNK, HIDDEN), jnp.float32) for _ in range(NBUF)],
          pltpu.VMEM_SHARED((NS, SBUF, CHUNK, HIDDEN), jnp.float32),
          pltpu.SemaphoreType.DMA,
          pltpu.SemaphoreType.DMA,
          pltpu.SemaphoreType.DMA,
      ],
  )
  def gather_kernel(idx_hbm, table_hbm, out_hbm, idx_v, bufs, spbuf, sem_g,
                    sem_c, sem_o):
    wid = lax.axis_index("s") * NC + lax.axis_index("c")
    sid = lax.axis_index("s")
    base = wid * B_PER_W
    pltpu.sync_copy(idx_hbm.at[pl.ds(base, B_PER_W)], idx_v)

    gathers = [None] * NCHUNKS
    copies = [None] * NCHUNKS
    outs = [None] * NCHUNKS

    def fire_gather(g):
      gathers[g] = pltpu.async_copy(
          table_hbm.at[idx_v.at[pl.ds(g * CHUNK, CHUNK)]], bufs[g % NBUF],
          sem_g)

    def fire_copy(g):
      copies[g] = pltpu.async_copy(
          bufs[g % NBUF], spbuf.at[sid].at[g % SBUF], sem_c)

    def fire_out(g):
      outs[g] = pltpu.async_copy(
          spbuf.at[sid].at[g % SBUF],
          out_hbm.at[pl.ds(base + g * CHUNK, CHUNK)], sem_o)

    for g in range(NBUF):
      fire_gather(g)
    for g in range(NCHUNKS):
      gathers[g].wait()
      if g >= SBUF:
        outs[g - SBUF].wait()
      fire_copy(g)
      copies[g].wait()
      fire_out(g)
      ng = g + NBUF
      if ng < NCHUNKS:
        fire_gather(ng)
    for g in range(max(0, NCHUNKS - SBUF), NCHUNKS):
      outs[g].wait()

  return gather_kernel


_gather = _make_kernel()


@jax.jit
def kernel(labels, table):
  return _gather(labels.astype(jnp.int32), table)


# speedup vs baseline: 1.0027x; 1.0027x over previous
import functools

import jax
import jax.numpy as jnp
from jax import lax
from jax.experimental import pallas as pl
from jax.experimental.pallas import tpu as pltpu
from jax.experimental.pallas import tpu_sc as plsc

NUM_CLASSES = 1000
HIDDEN = 1024
BATCH = 4096

NC = 2
NS = 16
NW = NC * NS
B_PER_W = BATCH // NW
CHUNK = 16
NCHUNKS = B_PER_W // CHUNK
NBUF = 7


def _make_kernel():
  mesh = plsc.VectorSubcoreMesh(
      core_axis_name="c", subcore_axis_name="s", num_cores=NC,
      num_subcores=NS)

  @functools.partial(
      pl.kernel,
      out_type=jax.ShapeDtypeStruct((BATCH, HIDDEN), jnp.float32),
      mesh=mesh,
      scratch_types=[
          pltpu.VMEM((B_PER_W,), jnp.int32),
          [pltpu.VMEM((CHUNK, HIDDEN), jnp.float32) for _ in range(NBUF)],
          pltpu.SemaphoreType.DMA,
          pltpu.SemaphoreType.DMA,
          pltpu.SemaphoreType.DMA,
      ],
  )
  def gather_kernel(idx_hbm, table_hbm, out_hbm, idx_v, bufs, sem_g, sem_o,
                    sem_i):
    wid = lax.axis_index("s") * NC + lax.axis_index("c")
    base = wid * B_PER_W
    first = pltpu.async_copy(
        idx_hbm.at[pl.ds(base, CHUNK)], idx_v.at[pl.ds(0, CHUNK)], sem_i)
    rest = pltpu.async_copy(
        idx_hbm.at[pl.ds(base + CHUNK, B_PER_W - CHUNK)],
        idx_v.at[pl.ds(CHUNK, B_PER_W - CHUNK)], sem_i)
    first.wait()

    gathers = [None] * NCHUNKS
    outs = [None] * NCHUNKS

    def fire_gather(g):
      gathers[g] = pltpu.async_copy(
          table_hbm.at[idx_v.at[pl.ds(g * CHUNK, CHUNK)]], bufs[g % NBUF],
          sem_g)

    def fire_out(g):
      outs[g] = pltpu.async_copy(
          bufs[g % NBUF], out_hbm.at[pl.ds(base + g * CHUNK, CHUNK)], sem_o)

    fire_gather(0)
    rest.wait()
    for g in range(1, min(NBUF, NCHUNKS)):
      fire_gather(g)
    for g in range(NCHUNKS):
      gathers[g].wait()
      fire_out(g)
      nxt = g + NBUF
      if nxt < NCHUNKS:
        outs[nxt - NBUF].wait()
        fire_gather(nxt)
    for g in range(max(0, NCHUNKS - NBUF), NCHUNKS):
      outs[g].wait()

  return gather_kernel


_gather = _make_kernel()


@jax.jit
def kernel(labels, table):
  return _gather(labels.astype(jnp.int32), table)


# speedup vs baseline: 1.0036x; 1.0009x over previous
import functools

import jax
import jax.numpy as jnp
from jax import lax
from jax.experimental import pallas as pl
from jax.experimental.pallas import tpu as pltpu
from jax.experimental.pallas import tpu_sc as plsc

NUM_CLASSES = 1000
HIDDEN = 1024
BATCH = 4096

NC = 2
NS = 16
NW = NC * NS
B_PER_W = BATCH // NW
CHUNK = 32
NCHUNKS = B_PER_W // CHUNK
NBUF = 3


def _make_kernel():
  mesh = plsc.VectorSubcoreMesh(
      core_axis_name="c", subcore_axis_name="s", num_cores=NC,
      num_subcores=NS)

  @functools.partial(
      pl.kernel,
      out_type=jax.ShapeDtypeStruct((BATCH, HIDDEN), jnp.float32),
      mesh=mesh,
      scratch_types=[
          pltpu.VMEM((B_PER_W,), jnp.int32),
          [pltpu.VMEM((CHUNK, HIDDEN), jnp.float32) for _ in range(NBUF)],
          pltpu.SemaphoreType.DMA,
          pltpu.SemaphoreType.DMA,
          pltpu.SemaphoreType.DMA,
      ],
  )
  def gather_kernel(idx_hbm, table_hbm, out_hbm, idx_v, bufs, sem_g, sem_o,
                    sem_i):
    wid = lax.axis_index("s") * NC + lax.axis_index("c")
    base = wid * B_PER_W
    first = pltpu.async_copy(
        idx_hbm.at[pl.ds(base, CHUNK)], idx_v.at[pl.ds(0, CHUNK)], sem_i)
    rest = pltpu.async_copy(
        idx_hbm.at[pl.ds(base + CHUNK, B_PER_W - CHUNK)],
        idx_v.at[pl.ds(CHUNK, B_PER_W - CHUNK)], sem_i)
    first.wait()

    gathers = [None] * NCHUNKS
    outs = [None] * NCHUNKS

    def fire_gather(g):
      gathers[g] = pltpu.async_copy(
          table_hbm.at[idx_v.at[pl.ds(g * CHUNK, CHUNK)]], bufs[g % NBUF],
          sem_g)

    def fire_out(g):
      outs[g] = pltpu.async_copy(
          bufs[g % NBUF], out_hbm.at[pl.ds(base + g * CHUNK, CHUNK)], sem_o)

    fire_gather(0)
    rest.wait()
    for g in range(1, min(NBUF, NCHUNKS)):
      fire_gather(g)
    for g in range(NCHUNKS):
      gathers[g].wait()
      fire_out(g)
      nxt = g + NBUF
      if nxt < NCHUNKS:
        outs[nxt - NBUF].wait()
        fire_gather(nxt)
    for g in range(max(0, NCHUNKS - NBUF), NCHUNKS):
      outs[g].wait()

  return gather_kernel


_gather = _make_kernel()


@jax.jit
def kernel(labels, table):
  return _gather(labels.astype(jnp.int32), table)
